# baseline (speedup 1.0000x reference)
import functools

import jax
import jax.numpy as jnp
from jax import lax
from jax.experimental import pallas as pl
from jax.experimental.pallas import tpu as pltpu
from jax.experimental.pallas import tpu_sc as plsc

_B = 16384
_D = 32
_NW = 32
_BPW = _B // _NW
_CH = 4
_CB = _BPW // _CH


def _take16(v, idx):
    dnums = lax.GatherDimensionNumbers(
        offset_dims=(), collapsed_slice_dims=(0,), start_index_map=(0,))
    return lax.gather(v, idx.reshape(16, 1), dnums, (1,),
                      mode=lax.GatherScatterMode.PROMISE_IN_BOUNDS)


def _gmf_body(uq_hbm, iq_hbm, us_hbm, is_hbm, par_hbm, utab_hbm, itab_hbm,
              out_hbm, uq_v, iq_v, us_v, is_v, urows_v, irows_v, par_v,
              out_v, sem):
    wid = lax.axis_index("s") * 2 + lax.axis_index("c")

    pltpu.sync_copy(uq_hbm.at[pl.ds(wid * _CH, _CH)], uq_v)
    pltpu.sync_copy(iq_hbm.at[pl.ds(wid * _CH, _CH)], iq_v)
    pltpu.sync_copy(us_hbm.at[pl.ds(wid * _CH, _CH)], us_v)
    pltpu.sync_copy(is_hbm.at[pl.ds(wid * _CH, _CH)], is_v)
    pltpu.sync_copy(par_hbm, par_v)

    iota = lax.iota(jnp.int32, 16)
    neg_b = par_v[pl.ds(_D, 16)]
    w_lo = par_v[pl.ds(0, 16)]
    w_hi = par_v[pl.ds(16, 16)]
    fifteen = jnp.full((16,), 15, jnp.int32)

    for c in range(_CH):
        cu = pltpu.async_copy(utab_hbm.at[uq_v.at[c]], urows_v, sem)
        ci = pltpu.async_copy(itab_hbm.at[iq_v.at[c]], irows_v, sem)
        cu.wait()
        ci.wait()

        def row_group(rg, carry):
            base = rg * 16
            acc = neg_b
            for j in range(16):
                r = base + j
                jj = jnp.full((16,), r, jnp.int32)
                sub = plsc.load_gather(us_v.at[c], [jj])
                col_lo = sub + iota
                col_hi = col_lo + 16
                u_lo = plsc.load_gather(urows_v, [jj, col_lo])
                u_hi = plsc.load_gather(urows_v, [jj, col_hi])
                sub_i = plsc.load_gather(is_v.at[c], [jj])
                icol_lo = sub_i + iota
                i_lo = plsc.load_gather(irows_v, [jj, icol_lo])
                i_hi = plsc.load_gather(irows_v, [jj, icol_lo + 16])
                s = u_lo * i_lo * w_lo + u_hi * i_hi * w_hi
                hs = _take16(plsc.cumsum(s), fifteen)
                acc = jnp.where(iota == j, hs + neg_b, acc)
            out_v[pl.ds(c * _CB + base, 16)] = 1.0 / (1.0 + jnp.exp(acc))
            return carry

        lax.fori_loop(0, _CB // 16, row_group, 0)

    pltpu.sync_copy(out_v, out_hbm.at[pl.ds(wid * _BPW, _BPW)])


def kernel(user_indices, item_indices, user_table, item_table, W, b):
    uidx = user_indices.astype(jnp.int32)
    iidx = item_indices.astype(jnp.int32)
    uq = (uidx >> 2).reshape(_NW * _CH, _CB)
    iq = (iidx >> 2).reshape(_NW * _CH, _CB)
    us = ((uidx & 3) * _D).reshape(_NW * _CH, _CB)
    i_s = ((iidx & 3) * _D).reshape(_NW * _CH, _CB)
    ut = user_table.reshape(250000, 128)
    it = item_table.reshape(250000, 128)
    params = jnp.concatenate(
        [-W.reshape(_D), jnp.broadcast_to(-b, (16,))]).astype(jnp.float32)

    mesh = plsc.VectorSubcoreMesh(core_axis_name="c", subcore_axis_name="s")
    run = functools.partial(
        pl.kernel, mesh=mesh,
        compiler_params=pltpu.CompilerParams(needs_layout_passes=False),
        out_type=jax.ShapeDtypeStruct((_B,), jnp.float32),
        scratch_types=[
            pltpu.VMEM((_CH, _CB), jnp.int32),
            pltpu.VMEM((_CH, _CB), jnp.int32),
            pltpu.VMEM((_CH, _CB), jnp.int32),
            pltpu.VMEM((_CH, _CB), jnp.int32),
            pltpu.VMEM((_CB, 128), jnp.float32),
            pltpu.VMEM((_CB, 128), jnp.float32),
            pltpu.VMEM((_D + 16,), jnp.float32),
            pltpu.VMEM((_BPW,), jnp.float32),
            pltpu.SemaphoreType.DMA,
        ],
    )(_gmf_body)
    out = run(uq, iq, us, i_s, params, ut, it)
    return out.reshape(_B, 1)
